# Initial kernel scaffold; baseline (speedup 1.0000x reference)
import jax
import jax.numpy as jnp
from jax.experimental import pallas as pl


def kernel(x, table):
    raise NotImplementedError("write your pallas kernel here")


# speedup vs baseline: 2.2249x; 2.2249x over previous
import functools

import jax
import jax.numpy as jnp
from jax import lax
from jax.experimental import pallas as pl
from jax.experimental.pallas import tpu as pltpu
from jax.experimental.pallas import tpu_sc as plsc

VOCAB = 1_000_000
D = 16
EPS = 1e-5

NC = 2
NS = 16
NW = NC * NS


_TC_BLOCK_ROWS = 1000


def _norm_body(e_ref, s_ref, o_ref):
    e = e_ref[...]
    s = s_ref[...]
    m = lax.dot(e, s, precision=lax.Precision.HIGHEST,
                preferred_element_type=jnp.float32)
    d = e - m
    v = lax.dot(d * d, s, precision=lax.Precision.HIGHEST,
                preferred_element_type=jnp.float32)
    o_ref[...] = d * lax.rsqrt(v + EPS)


def _normalize_table(table):
    rows = VOCAB * D // 128
    t2 = table.reshape(rows, 128)
    s = jnp.kron(jnp.eye(128 // D, dtype=jnp.float32),
                 jnp.full((D, D), 1.0 / D, dtype=jnp.float32))
    out = pl.pallas_call(
        _norm_body,
        grid=(rows // _TC_BLOCK_ROWS,),
        in_specs=[
            pl.BlockSpec((_TC_BLOCK_ROWS, 128), lambda i: (i, 0)),
            pl.BlockSpec((128, 128), lambda i: (0, 0)),
        ],
        out_specs=pl.BlockSpec((_TC_BLOCK_ROWS, 128), lambda i: (i, 0)),
        out_shape=jax.ShapeDtypeStruct((rows, 128), jnp.float32),
    )(t2, s)
    return out.reshape(VOCAB, D)


_CHUNK = 16


def _make_gather(n_idx_rows):
    rows_per_w = n_idx_rows // NW
    n_steps = rows_per_w // _CHUNK
    mesh = plsc.VectorSubcoreMesh(core_axis_name="c", subcore_axis_name="s",
                                  num_cores=NC, num_subcores=NS)

    @functools.partial(
        pl.kernel,
        out_type=jax.ShapeDtypeStruct((n_idx_rows * 128, D), jnp.float32),
        mesh=mesh,
        scratch_types=[
            pltpu.VMEM((_CHUNK, 128), jnp.int32),
            pltpu.VMEM((_CHUNK * 128, D), jnp.float32),
            pltpu.SemaphoreType.DMA,
        ],
        compiler_params=pltpu.CompilerParams(use_tc_tiling_on_sc=False),
    )
    def gather_k(tab_hbm, idx_hbm, out_hbm, idx_v, rows_v, sem):
        wid = lax.axis_index("s") * NC + lax.axis_index("c")
        wrow = wid * rows_per_w

        def step(j, carry):
            roff = wrow + j * _CHUNK
            pltpu.sync_copy(idx_hbm.at[pl.ds(roff, _CHUNK)], idx_v)
            copies = [
                pltpu.async_copy(tab_hbm.at[idx_v.at[t]],
                                 rows_v.at[pl.ds(t * 128, 128)], sem)
                for t in range(_CHUNK)
            ]
            for c in copies:
                c.wait()
            pltpu.sync_copy(rows_v, out_hbm.at[pl.ds(roff * 128, _CHUNK * 128)])
            return carry

        lax.fori_loop(0, n_steps, step, 0)

    return gather_k


def kernel(x, table):
    b, h = x.shape
    n_idx_rows = b * h // 128
    ntab = _normalize_table(table)
    idx2 = x.reshape(n_idx_rows, 128)
    out = _make_gather(n_idx_rows)(ntab, idx2)
    return out.reshape(b, h, D)


# speedup vs baseline: 2.2527x; 1.0125x over previous
import functools

import jax
import jax.numpy as jnp
from jax import lax
from jax.experimental import pallas as pl
from jax.experimental.pallas import tpu as pltpu
from jax.experimental.pallas import tpu_sc as plsc

VOCAB = 1_000_000
D = 16
EPS = 1e-5

NC = 2
NS = 16
NW = NC * NS


_TC_BLOCK_ROWS = 1000


def _norm_body(e_ref, s_ref, o_ref):
    e = e_ref[...]
    s = s_ref[...]
    m = lax.dot(e, s, precision=lax.Precision.HIGHEST,
                preferred_element_type=jnp.float32)
    d = e - m
    v = lax.dot(d * d, s, precision=lax.Precision.HIGHEST,
                preferred_element_type=jnp.float32)
    o_ref[...] = d * lax.rsqrt(v + EPS)


def _normalize_table(table):
    rows = VOCAB * D // 128
    t2 = table.reshape(rows, 128)
    s = jnp.kron(jnp.eye(128 // D, dtype=jnp.float32),
                 jnp.full((D, D), 1.0 / D, dtype=jnp.float32))
    out = pl.pallas_call(
        _norm_body,
        grid=(rows // _TC_BLOCK_ROWS,),
        in_specs=[
            pl.BlockSpec((_TC_BLOCK_ROWS, 128), lambda i: (i, 0)),
            pl.BlockSpec((128, 128), lambda i: (0, 0)),
        ],
        out_specs=pl.BlockSpec((_TC_BLOCK_ROWS, 128), lambda i: (i, 0)),
        out_shape=jax.ShapeDtypeStruct((rows, 128), jnp.float32),
    )(t2, s)
    return out.reshape(VOCAB, D)


_STEP_ROWS = 4


def _make_gather(n_idx_rows):
    rows_per_w = n_idx_rows // NW
    n_steps = rows_per_w // (2 * _STEP_ROWS)
    mesh = plsc.VectorSubcoreMesh(core_axis_name="c", subcore_axis_name="s",
                                  num_cores=NC, num_subcores=NS)
    step_idx = _STEP_ROWS * 128

    @functools.partial(
        pl.kernel,
        out_type=jax.ShapeDtypeStruct((n_idx_rows * 128, D), jnp.float32),
        mesh=mesh,
        scratch_types=[
            pltpu.VMEM((rows_per_w, 128), jnp.int32),
            pltpu.VMEM((step_idx, D), jnp.float32),
            pltpu.VMEM((step_idx, D), jnp.float32),
            pltpu.SemaphoreType.DMA,
            pltpu.SemaphoreType.DMA,
            pltpu.SemaphoreType.DMA,
            pltpu.SemaphoreType.DMA,
        ],
        compiler_params=pltpu.CompilerParams(use_tc_tiling_on_sc=False),
    )
    def gather_k(tab_hbm, idx_hbm, out_hbm, idx_v, rows0, rows1, sg0, sg1,
                 ss0, ss1):
        wid = lax.axis_index("s") * NC + lax.axis_index("c")
        wrow = wid * rows_per_w
        pltpu.sync_copy(idx_hbm.at[pl.ds(wrow, rows_per_w)], idx_v)

        def fire(s, rows_v, sem):
            for r in range(_STEP_ROWS):
                pltpu.async_copy(tab_hbm.at[idx_v.at[s * _STEP_ROWS + r]],
                                 rows_v.at[pl.ds(r * 128, 128)], sem)

        def drain_gather(rows_v, sem):
            for r in range(_STEP_ROWS):
                pltpu.make_async_copy(tab_hbm.at[idx_v.at[r]],
                                      rows_v.at[pl.ds(r * 128, 128)],
                                      sem).wait()

        def out_slice(s):
            return out_hbm.at[pl.ds((wrow + s * _STEP_ROWS) * 128, step_idx)]

        def drain_store(rows_v, sem):
            pltpu.make_async_copy(rows_v, out_slice(0), sem).wait()

        def body(k, carry):
            s0 = 2 * k

            @pl.when(k > 0)
            def _():
                drain_store(rows0, ss0)

            fire(s0, rows0, sg0)

            @pl.when(k > 0)
            def _():
                drain_store(rows1, ss1)

            fire(s0 + 1, rows1, sg1)
            drain_gather(rows0, sg0)
            pltpu.async_copy(rows0, out_slice(s0), ss0)
            drain_gather(rows1, sg1)
            pltpu.async_copy(rows1, out_slice(s0 + 1), ss1)
            return carry

        lax.fori_loop(0, n_steps, body, 0)
        drain_store(rows0, ss0)
        drain_store(rows1, ss1)

    return gather_k


def kernel(x, table):
    b, h = x.shape
    n_idx_rows = b * h // 128
    ntab = _normalize_table(table)
    idx2 = x.reshape(n_idx_rows, 128)
    out = _make_gather(n_idx_rows)(ntab, idx2)
    return out.reshape(b, h, D)


# speedup vs baseline: 3.4255x; 1.5207x over previous
import functools

import jax
import jax.numpy as jnp
from jax import lax
from jax.experimental import pallas as pl
from jax.experimental.pallas import tpu as pltpu
from jax.experimental.pallas import tpu_sc as plsc

VOCAB = 1_000_000
D = 16
EPS = 1e-5

B = 16384
H = 200

NC = 2
NS = 16
NW = NC * NS

BPW = B // NW
NBT = BPW // 128
HT = H // 8


_TC_BLOCK_ROWS = 1000


def _norm_body(e_ref, s_ref, o_ref):
    e = e_ref[...]
    s = s_ref[...]
    m = lax.dot(e, s, precision=lax.Precision.HIGHEST,
                preferred_element_type=jnp.float32)
    d = e - m
    v = lax.dot(d * d, s, precision=lax.Precision.HIGHEST,
                preferred_element_type=jnp.float32)
    o_ref[...] = d * lax.rsqrt(v + EPS)


def _normalize_table(table):
    rows = VOCAB * D // 128
    t2 = table.reshape(rows, 128)
    s = jnp.kron(jnp.eye(128 // D, dtype=jnp.float32),
                 jnp.full((D, D), 1.0 / D, dtype=jnp.float32))
    out = pl.pallas_call(
        _norm_body,
        grid=(rows // _TC_BLOCK_ROWS,),
        in_specs=[
            pl.BlockSpec((_TC_BLOCK_ROWS, 128), lambda i: (i, 0)),
            pl.BlockSpec((128, 128), lambda i: (0, 0)),
        ],
        out_specs=pl.BlockSpec((_TC_BLOCK_ROWS, 128), lambda i: (i, 0)),
        out_shape=jax.ShapeDtypeStruct((rows, 128), jnp.float32),
    )(t2, s)
    return out.reshape(VOCAB, D)


_N_STEPS = H


def _make_gather():
    mesh = plsc.VectorSubcoreMesh(core_axis_name="c", subcore_axis_name="s",
                                  num_cores=NC, num_subcores=NS)

    @functools.partial(
        pl.kernel,
        out_type=jax.ShapeDtypeStruct((B * H * D // 128, 128), jnp.float32),
        mesh=mesh,
        scratch_types=[
            pltpu.VMEM((NBT * 8, 128), jnp.int32),
            pltpu.VMEM((BPW, D), jnp.float32),
            pltpu.VMEM((BPW, D), jnp.float32),
            pltpu.VMEM((2, NBT * 8, 128), jnp.float32),
            pltpu.VMEM((2, NBT * 8, 128), jnp.float32),
            pltpu.SemaphoreType.DMA,
            pltpu.SemaphoreType.DMA,
            pltpu.SemaphoreType.DMA,
            pltpu.SemaphoreType.DMA,
        ],
        compiler_params=pltpu.CompilerParams(use_tc_tiling_on_sc=False,
                                             needs_layout_passes=False),
    )
    def gather_k(tab_hbm, q_hbm, p_hbm, idx_v, rows0, rows1, st0, st1,
                 sg0, sg1, ss0, ss1):
        wid = lax.axis_index("s") * NC + lax.axis_index("c")
        rows_b = (rows0, rows1)
        stage_b = (st0, st1)
        sg_b = (sg0, sg1)
        ss_b = (ss0, ss1)

        iota = lax.iota(jnp.int32, 16)
        dt_vec = lax.shift_right_logical(iota, 3)
        ds_vec = lax.bitwise_and(iota, jnp.int32(7))

        def load_idx_block(ht):
            pltpu.sync_copy(q_hbm.at[pl.ds(ht * 1024 + wid * (NBT * 8),
                                           NBT * 8)], idx_v)

        def fire(s, a):
            hs1 = lax.rem(s, 8)
            for c in range(NBT):
                pltpu.async_copy(tab_hbm.at[idx_v.at[c * 8 + hs1]],
                                 rows_b[a].at[pl.ds(c * 128, 128)], sg_b[a])

        def wait_gather(a):
            for c in range(NBT):
                pltpu.make_async_copy(tab_hbm.at[idx_v.at[0]],
                                      rows_b[a].at[pl.ds(c * 128, 128)],
                                      sg_b[a]).wait()

        def wait_store(a):
            for dt in range(2):
                pltpu.make_async_copy(stage_b[a].at[dt],
                                      p_hbm.at[pl.ds(0, NBT * 8)],
                                      ss_b[a]).wait()

        def transpose(a):
            rows_v = rows_b[a]
            stage = stage_b[a]
            for btl in range(NBT):
                row32 = ds_vec + btl * 8

                def tbody(g, carry):
                    base = btl * 128 + g * 8
                    for u in range(8):
                        r = base + u
                        v = rows_v[r, :]
                        bl_vec = jnp.full((16,), 0, jnp.int32) + (r - btl * 128)
                        plsc.store_scatter(stage, [dt_vec, row32, bl_vec], v)
                    return carry

                lax.fori_loop(0, 16, tbody, 0)

        def store(s, a):
            for dt in range(2):
                pltpu.async_copy(
                    stage_b[a].at[dt],
                    p_hbm.at[pl.ds(s * 2048 + dt * 1024 + wid * (NBT * 8),
                                   NBT * 8)],
                    ss_b[a])

        load_idx_block(0)
        fire(0, 0)

        def body(k, carry):
            for u in range(2):
                s = 2 * k + u
                a = u
                hs = lax.rem(s, 8)

                @pl.when(jnp.logical_and(s < _N_STEPS - 1, hs != 7))
                def _():
                    fire(s + 1, 1 - a)

                wait_gather(a)

                @pl.when(jnp.logical_and(s < _N_STEPS - 1, hs == 7))
                def _():
                    load_idx_block((s + 1) // 8)
                    fire(s + 1, 1 - a)

                @pl.when(s >= 2)
                def _():
                    wait_store(a)

                transpose(a)
                store(s, a)
            return carry

        lax.fori_loop(0, _N_STEPS // 2, body, 0)
        wait_store(0)
        wait_store(1)

    return gather_k


def kernel(x, table):
    ntab = _normalize_table(table)
    q = (x.T.reshape(HT, 8, 128, 128).transpose(0, 2, 1, 3)
         .reshape(HT * 1024, 128))
    p = _make_gather()(ntab, q)
    out = (p.reshape(H, 2, 128, 8, 128).transpose(2, 4, 0, 1, 3)
           .reshape(B, H, D))
    return out


# speedup vs baseline: 4.9442x; 1.4433x over previous
import functools

import jax
import jax.numpy as jnp
from jax import lax
from jax.experimental import pallas as pl
from jax.experimental.pallas import tpu as pltpu
from jax.experimental.pallas import tpu_sc as plsc

VOCAB = 1_000_000
D = 16
EPS = 1e-5

B = 16384
H = 200

NC = 2
NS = 16
NW = NC * NS

BPW = B // NW
NBT = BPW // 128
HT = H // 8


_TC_BLOCK_ROWS = 1000


def _norm_body(e_ref, s_ref, o_ref):
    e = e_ref[...]
    s = s_ref[...]
    m = lax.dot(e, s, precision=lax.Precision.HIGHEST,
                preferred_element_type=jnp.float32)
    d = e - m
    v = lax.dot(d * d, s, precision=lax.Precision.HIGHEST,
                preferred_element_type=jnp.float32)
    o_ref[...] = d * lax.rsqrt(v + EPS)


def _normalize_table(table):
    rows = VOCAB * D // 128
    t2 = table.reshape(rows, 128)
    s = jnp.kron(jnp.eye(128 // D, dtype=jnp.float32),
                 jnp.full((D, D), 1.0 / D, dtype=jnp.float32))
    out = pl.pallas_call(
        _norm_body,
        grid=(rows // _TC_BLOCK_ROWS,),
        in_specs=[
            pl.BlockSpec((_TC_BLOCK_ROWS, 128), lambda i: (i, 0)),
            pl.BlockSpec((128, 128), lambda i: (0, 0)),
        ],
        out_specs=pl.BlockSpec((_TC_BLOCK_ROWS, 128), lambda i: (i, 0)),
        out_shape=jax.ShapeDtypeStruct((rows, 128), jnp.float32),
    )(t2, s)
    return out.reshape(VOCAB, D)


_N_STEPS = H


def _make_gather():
    mesh = plsc.VectorSubcoreMesh(core_axis_name="c", subcore_axis_name="s",
                                  num_cores=NC, num_subcores=NS)

    @functools.partial(
        pl.kernel,
        out_type=jax.ShapeDtypeStruct((B * H * D // 128, 128), jnp.float32),
        mesh=mesh,
        scratch_types=[
            pltpu.VMEM((NBT * 8, 128), jnp.int32),
            pltpu.VMEM((BPW, D), jnp.float32),
            pltpu.VMEM((BPW, D), jnp.float32),
            pltpu.VMEM((2, 40, 129), jnp.float32),
            pltpu.VMEM((2, 40, 129), jnp.float32),
            pltpu.SemaphoreType.DMA,
            pltpu.SemaphoreType.DMA,
            pltpu.SemaphoreType.DMA,
            pltpu.SemaphoreType.DMA,
        ],
        compiler_params=pltpu.CompilerParams(use_tc_tiling_on_sc=False,
                                             needs_layout_passes=False),
    )
    def gather_k(tab_hbm, q_hbm, p_hbm, idx_v, rows0, rows1, st0, st1,
                 sg0, sg1, ss0, ss1):
        wid = lax.axis_index("s") * NC + lax.axis_index("c")
        rows_b = (rows0, rows1)
        stage_b = (st0, st1)
        sg_b = (sg0, sg1)
        ss_b = (ss0, ss1)

        iota = lax.iota(jnp.int32, 16)
        dt_vec = lax.shift_right_logical(iota, 3)
        ds_vec = lax.bitwise_and(iota, jnp.int32(7))

        def load_idx_block(ht):
            pltpu.sync_copy(q_hbm.at[pl.ds(ht * 1024 + wid * (NBT * 8),
                                           NBT * 8)], idx_v)

        def fire(s, a):
            hs1 = lax.rem(s, 8)
            for c in range(NBT):
                pltpu.async_copy(tab_hbm.at[idx_v.at[c * 8 + hs1]],
                                 rows_b[a].at[pl.ds(c * 128, 128)], sg_b[a])

        def wait_gather(a):
            for c in range(NBT):
                pltpu.make_async_copy(tab_hbm.at[idx_v.at[0]],
                                      rows_b[a].at[pl.ds(c * 128, 128)],
                                      sg_b[a]).wait()

        def wait_store(a):
            for dt in range(2):
                pltpu.make_async_copy(
                    stage_b[a].at[dt, pl.ds(0, NBT * 8), pl.ds(0, 128)],
                    p_hbm.at[pl.ds(0, NBT * 8)],
                    ss_b[a]).wait()

        def transpose(a):
            rows_v = rows_b[a]
            stage = stage_b[a]
            for btl in range(NBT):
                row32 = ds_vec + btl * 8

                def tbody(g, carry):
                    base = btl * 128 + g * 8
                    for u in range(8):
                        r = base + u
                        v = rows_v[r, :]
                        bl_vec = jnp.full((16,), 0, jnp.int32) + (r - btl * 128)
                        plsc.store_scatter(stage, [dt_vec, row32, bl_vec], v)
                    return carry

                lax.fori_loop(0, 16, tbody, 0)

        def store(s, a):
            for dt in range(2):
                pltpu.async_copy(
                    stage_b[a].at[dt, pl.ds(0, NBT * 8), pl.ds(0, 128)],
                    p_hbm.at[pl.ds(s * 2048 + dt * 1024 + wid * (NBT * 8),
                                   NBT * 8)],
                    ss_b[a])

        load_idx_block(0)
        fire(0, 0)

        def body(k, carry):
            for u in range(2):
                s = 2 * k + u
                a = u
                hs = lax.rem(s, 8)

                @pl.when(jnp.logical_and(s < _N_STEPS - 1, hs != 7))
                def _():
                    fire(s + 1, 1 - a)

                wait_gather(a)

                @pl.when(jnp.logical_and(s < _N_STEPS - 1, hs == 7))
                def _():
                    load_idx_block((s + 1) // 8)
                    fire(s + 1, 1 - a)

                @pl.when(s >= 2)
                def _():
                    wait_store(a)

                transpose(a)
                store(s, a)
            return carry

        lax.fori_loop(0, _N_STEPS // 2, body, 0)
        wait_store(0)
        wait_store(1)

    return gather_k


def kernel(x, table):
    ntab = _normalize_table(table)
    q = (x.T.reshape(HT, 8, 128, 128).transpose(0, 2, 1, 3)
         .reshape(HT * 1024, 128))
    p = _make_gather()(ntab, q)
    out = (p.reshape(H, 2, 128, 8, 128).transpose(2, 4, 0, 1, 3)
           .reshape(B, H, D))
    return out


# speedup vs baseline: 6.7119x; 1.3575x over previous
import functools

import jax
import jax.numpy as jnp
from jax import lax
from jax.experimental import pallas as pl
from jax.experimental.pallas import tpu as pltpu
from jax.experimental.pallas import tpu_sc as plsc

VOCAB = 1_000_000
D = 16
EPS = 1e-5

B = 16384
H = 200

NC = 2
NS = 16
NW = NC * NS

BPW = B // NW
NBT = BPW // 128
HT = H // 8


_TC_BLOCK_ROWS = 5000


def _norm_body(e_ref, s_ref, o_ref):
    e = e_ref[...]
    s = s_ref[...]
    m = lax.dot(e, s, preferred_element_type=jnp.float32)
    d = e - m
    v = lax.dot(d * d, s, preferred_element_type=jnp.float32)
    o_ref[...] = d * lax.rsqrt(v + EPS)


def _normalize_table(table):
    rows = VOCAB * D // 128
    t2 = table.reshape(rows, 128)
    s = jnp.kron(jnp.eye(128 // D, dtype=jnp.float32),
                 jnp.full((D, D), 1.0 / D, dtype=jnp.float32))
    out = pl.pallas_call(
        _norm_body,
        grid=(rows // _TC_BLOCK_ROWS,),
        in_specs=[
            pl.BlockSpec((_TC_BLOCK_ROWS, 128), lambda i: (i, 0)),
            pl.BlockSpec((128, 128), lambda i: (0, 0)),
        ],
        out_specs=pl.BlockSpec((_TC_BLOCK_ROWS, 128), lambda i: (i, 0)),
        out_shape=jax.ShapeDtypeStruct((rows, 128), jnp.float32),
    )(t2, s)
    return out.reshape(VOCAB, D)


_N_STEPS = H


def _make_gather():
    mesh = plsc.VectorSubcoreMesh(core_axis_name="c", subcore_axis_name="s",
                                  num_cores=NC, num_subcores=NS)

    @functools.partial(
        pl.kernel,
        out_type=jax.ShapeDtypeStruct((B * H * D // 128, 128), jnp.float32),
        mesh=mesh,
        scratch_types=[
            pltpu.VMEM((NBT * 8, 128), jnp.int32),
            pltpu.VMEM((BPW, D), jnp.float32),
            pltpu.VMEM((BPW, D), jnp.float32),
            pltpu.VMEM((2, 40, 129), jnp.float32),
            pltpu.VMEM((2, 40, 129), jnp.float32),
            pltpu.SemaphoreType.DMA,
            pltpu.SemaphoreType.DMA,
            pltpu.SemaphoreType.DMA,
            pltpu.SemaphoreType.DMA,
        ],
        compiler_params=pltpu.CompilerParams(use_tc_tiling_on_sc=False,
                                             needs_layout_passes=False),
    )
    def gather_k(tab_hbm, q_hbm, p_hbm, idx_v, rows0, rows1, st0, st1,
                 sg0, sg1, ss0, ss1):
        wid = lax.axis_index("s") * NC + lax.axis_index("c")
        rows_b = (rows0, rows1)
        stage_b = (st0, st1)
        sg_b = (sg0, sg1)
        ss_b = (ss0, ss1)

        iota = lax.iota(jnp.int32, 16)
        dt_vec = lax.shift_right_logical(iota, 3)
        ds_vec = lax.bitwise_and(iota, jnp.int32(7))

        def load_idx_block(ht):
            pltpu.sync_copy(q_hbm.at[pl.ds(ht * 1024 + wid * (NBT * 8),
                                           NBT * 8)], idx_v)

        def fire(s, a):
            hs1 = lax.rem(s, 8)
            for c in range(NBT):
                pltpu.async_copy(tab_hbm.at[idx_v.at[c * 8 + hs1]],
                                 rows_b[a].at[pl.ds(c * 128, 128)], sg_b[a])

        def wait_gather(a):
            for c in range(NBT):
                pltpu.make_async_copy(tab_hbm.at[idx_v.at[0]],
                                      rows_b[a].at[pl.ds(c * 128, 128)],
                                      sg_b[a]).wait()

        def wait_store(a):
            for dt in range(2):
                pltpu.make_async_copy(
                    stage_b[a].at[dt, pl.ds(0, NBT * 8), pl.ds(0, 128)],
                    p_hbm.at[pl.ds(0, NBT * 8)],
                    ss_b[a]).wait()

        def transpose(a):
            rows_v = rows_b[a]
            stage = stage_b[a]
            for btl in range(NBT):
                row32 = ds_vec + btl * 8

                def tbody(g, carry):
                    base = btl * 128 + g * 8
                    for u in range(8):
                        r = base + u
                        v = rows_v[r, :]
                        bl_vec = jnp.full((16,), 0, jnp.int32) + (r - btl * 128)
                        plsc.store_scatter(stage, [dt_vec, row32, bl_vec], v)
                    return carry

                lax.fori_loop(0, 16, tbody, 0)

        def store(s, a):
            for dt in range(2):
                pltpu.async_copy(
                    stage_b[a].at[dt, pl.ds(0, NBT * 8), pl.ds(0, 128)],
                    p_hbm.at[pl.ds(s * 2048 + dt * 1024 + wid * (NBT * 8),
                                   NBT * 8)],
                    ss_b[a])

        load_idx_block(0)
        fire(0, 0)

        def body(k, carry):
            for u in range(2):
                s = 2 * k + u
                a = u
                hs = lax.rem(s, 8)

                @pl.when(jnp.logical_and(s < _N_STEPS - 1, hs != 7))
                def _():
                    fire(s + 1, 1 - a)

                wait_gather(a)

                @pl.when(jnp.logical_and(s < _N_STEPS - 1, hs == 7))
                def _():
                    load_idx_block((s + 1) // 8)
                    fire(s + 1, 1 - a)

                @pl.when(s >= 2)
                def _():
                    wait_store(a)

                transpose(a)
                store(s, a)
            return carry

        lax.fori_loop(0, _N_STEPS // 2, body, 0)
        wait_store(0)
        wait_store(1)

    return gather_k


def kernel(x, table):
    ntab = _normalize_table(table)
    q = (x.T.reshape(HT, 8, 128, 128).transpose(0, 2, 1, 3)
         .reshape(HT * 1024, 128))
    p = _make_gather()(ntab, q)
    out = (p.reshape(H, 2, 128, 8, 128).transpose(2, 4, 0, 1, 3)
           .reshape(B, H, D))
    return out


# speedup vs baseline: 7.9459x; 1.1839x over previous
import functools

import jax
import jax.numpy as jnp
from jax import lax
from jax.experimental import pallas as pl
from jax.experimental.pallas import tpu as pltpu
from jax.experimental.pallas import tpu_sc as plsc

VOCAB = 1_000_000
D = 16
EPS = 1e-5

B = 16384
H = 200

NC = 2
NS = 16
NW = NC * NS

BPW = B // NW
NBT = BPW // 128
HT = H // 8


_TC_BLOCK_W = 65536
VP = 1_048_576
VPX = VP // 128


_TC_BLOCK_K = _TC_BLOCK_W // 128


def _norm_body(t_ref, o_ref):
    tt = t_ref[...]
    m = jnp.mean(tt, axis=0, keepdims=True)
    d = tt - m
    v = jnp.mean(d * d, axis=0, keepdims=True)
    r = d * lax.rsqrt(v + EPS)
    o_ref[...] = r.reshape(2, 8, _TC_BLOCK_K, 128).swapaxes(1, 2)


def _normalize_table(table):
    grid = (VP + _TC_BLOCK_W - 1) // _TC_BLOCK_W
    return pl.pallas_call(
        _norm_body,
        grid=(grid,),
        in_specs=[pl.BlockSpec((D, _TC_BLOCK_W), lambda i: (0, i))],
        out_specs=pl.BlockSpec((2, _TC_BLOCK_K, 8, 128),
                               lambda i: (0, i, 0, 0)),
        out_shape=jax.ShapeDtypeStruct((2, VPX, 8, 128), jnp.float32),
    )(table.T)


VPT = VP // NW
TCH = 1024
NCH = VPT // TCH


def _make_prep():
    mesh = plsc.VectorSubcoreMesh(core_axis_name="c", subcore_axis_name="s",
                                  num_cores=NC, num_subcores=NS)

    @functools.partial(
        pl.kernel,
        out_type=jax.ShapeDtypeStruct((VP, D), jnp.float32),
        mesh=mesh,
        scratch_types=[
            pltpu.VMEM((2, 8, 8, 128), jnp.float32),
            pltpu.VMEM((2, 8, 8, 128), jnp.float32),
            pltpu.VMEM((TCH, 17), jnp.float32),
            pltpu.VMEM((TCH, 17), jnp.float32),
            pltpu.SemaphoreType.DMA,
            pltpu.SemaphoreType.DMA,
            pltpu.SemaphoreType.DMA,
            pltpu.SemaphoreType.DMA,
        ],
        compiler_params=pltpu.CompilerParams(use_tc_tiling_on_sc=False,
                                             needs_layout_passes=False),
    )
    def prep_k(ntp_hbm, out_hbm, in0, in1, ob0, ob1, si0, si1, so0, so1):
        wid = lax.axis_index("s") * NC + lax.axis_index("c")
        v0 = wid * VPT
        vt0 = v0 // 128
        in_b = (in0, in1)
        ob_b = (ob0, ob1)
        si_b = (si0, si1)
        so_b = (so0, so1)
        iota = lax.iota(jnp.int32, 16)
        col_d = [jnp.full((16,), 0, jnp.int32) + d for d in range(D)]

        def load(k, b):
            pltpu.async_copy(ntp_hbm.at[:, pl.ds(vt0 + k * 8, 8)],
                             in_b[b], si_b[b])

        def wait_load(b):
            pltpu.make_async_copy(ntp_hbm.at[:, pl.ds(0, 8)], in_b[b],
                                  si_b[b]).wait()

        def store(k, b):
            pltpu.async_copy(ob_b[b].at[:, pl.ds(0, D)],
                             out_hbm.at[pl.ds(v0 + k * TCH, TCH)], so_b[b])

        def wait_store(b):
            pltpu.make_async_copy(ob_b[b].at[:, pl.ds(0, D)],
                                  out_hbm.at[pl.ds(0, TCH)], so_b[b]).wait()

        def transpose_chunk(b):
            inb = in_b[b]
            ob = ob_b[b]

            def gbody(m, carry):
                vtl = lax.div(m, 8)
                g = lax.rem(m, 8)
                row_v = iota + m * 16
                for dt in range(2):
                    for ds in range(8):
                        v = inb[dt, vtl, ds, pl.ds(g * 16, 16)]
                        plsc.store_scatter(ob, [row_v, col_d[dt * 8 + ds]], v)
                return carry

            lax.fori_loop(0, TCH // 16, gbody, 0)

        load(0, 0)

        def body(kp, carry):
            for u in range(2):
                k = 2 * kp + u
                b = u
                wait_load(b)

                @pl.when(k < NCH - 1)
                def _():
                    load(k + 1, 1 - b)

                @pl.when(k >= 2)
                def _():
                    wait_store(b)

                transpose_chunk(b)
                store(k, b)
            return carry

        lax.fori_loop(0, NCH // 2, body, 0)
        wait_store(0)
        wait_store(1)

    return prep_k


_N_STEPS = H


def _make_gather():
    mesh = plsc.VectorSubcoreMesh(core_axis_name="c", subcore_axis_name="s",
                                  num_cores=NC, num_subcores=NS)

    @functools.partial(
        pl.kernel,
        out_type=jax.ShapeDtypeStruct((B * H * D // 128, 128), jnp.float32),
        mesh=mesh,
        scratch_types=[
            pltpu.VMEM((NBT * 8, 128), jnp.int32),
            pltpu.VMEM((BPW, D), jnp.float32),
            pltpu.VMEM((BPW, D), jnp.float32),
            pltpu.VMEM((2, 40, 129), jnp.float32),
            pltpu.VMEM((2, 40, 129), jnp.float32),
            pltpu.SemaphoreType.DMA,
            pltpu.SemaphoreType.DMA,
            pltpu.SemaphoreType.DMA,
            pltpu.SemaphoreType.DMA,
        ],
        compiler_params=pltpu.CompilerParams(use_tc_tiling_on_sc=False,
                                             needs_layout_passes=False),
    )
    def gather_k(tab_hbm, q_hbm, p_hbm, idx_v, rows0, rows1, st0, st1,
                 sg0, sg1, ss0, ss1):
        wid = lax.axis_index("s") * NC + lax.axis_index("c")
        rows_b = (rows0, rows1)
        stage_b = (st0, st1)
        sg_b = (sg0, sg1)
        ss_b = (ss0, ss1)

        iota = lax.iota(jnp.int32, 16)
        dt_vec = lax.shift_right_logical(iota, 3)
        ds_vec = lax.bitwise_and(iota, jnp.int32(7))

        def load_idx_block(ht):
            pltpu.sync_copy(q_hbm.at[pl.ds(ht * 1024 + wid * (NBT * 8),
                                           NBT * 8)], idx_v)

        def fire(s, a):
            hs1 = lax.rem(s, 8)
            for c in range(NBT):
                pltpu.async_copy(tab_hbm.at[idx_v.at[c * 8 + hs1]],
                                 rows_b[a].at[pl.ds(c * 128, 128)], sg_b[a])

        def wait_gather(a):
            for c in range(NBT):
                pltpu.make_async_copy(tab_hbm.at[idx_v.at[0]],
                                      rows_b[a].at[pl.ds(c * 128, 128)],
                                      sg_b[a]).wait()

        def wait_store(a):
            for dt in range(2):
                pltpu.make_async_copy(
                    stage_b[a].at[dt, pl.ds(0, NBT * 8), pl.ds(0, 128)],
                    p_hbm.at[pl.ds(0, NBT * 8)],
                    ss_b[a]).wait()

        def transpose(a):
            rows_v = rows_b[a]
            stage = stage_b[a]
            for btl in range(NBT):
                row32 = ds_vec + btl * 8

                def tbody(g, carry):
                    base = btl * 128 + g * 8
                    for u in range(8):
                        r = base + u
                        v = rows_v[r, :]
                        bl_vec = jnp.full((16,), 0, jnp.int32) + (r - btl * 128)
                        plsc.store_scatter(stage, [dt_vec, row32, bl_vec], v)
                    return carry

                lax.fori_loop(0, 16, tbody, 0)

        def store(s, a):
            for dt in range(2):
                pltpu.async_copy(
                    stage_b[a].at[dt, pl.ds(0, NBT * 8), pl.ds(0, 128)],
                    p_hbm.at[pl.ds(s * 2048 + dt * 1024 + wid * (NBT * 8),
                                   NBT * 8)],
                    ss_b[a])

        load_idx_block(0)
        fire(0, 0)

        def body(k, carry):
            for u in range(2):
                s = 2 * k + u
                a = u
                hs = lax.rem(s, 8)

                @pl.when(jnp.logical_and(s < _N_STEPS - 1, hs != 7))
                def _():
                    fire(s + 1, 1 - a)

                wait_gather(a)

                @pl.when(jnp.logical_and(s < _N_STEPS - 1, hs == 7))
                def _():
                    load_idx_block((s + 1) // 8)
                    fire(s + 1, 1 - a)

                @pl.when(s >= 2)
                def _():
                    wait_store(a)

                transpose(a)
                store(s, a)
            return carry

        lax.fori_loop(0, _N_STEPS // 2, body, 0)
        wait_store(0)
        wait_store(1)

    return gather_k


def kernel(x, table):
    ntab = _make_prep()(_normalize_table(table))
    q = (x.T.reshape(HT, 8, 128, 128).transpose(0, 2, 1, 3)
         .reshape(HT * 1024, 128))
    p = _make_gather()(ntab, q)
    out = (p.reshape(H, 2, 128, 8, 128).transpose(2, 4, 0, 1, 3)
           .reshape(B, H, D))
    return out


# speedup vs baseline: 8.1578x; 1.0267x over previous
import functools

import jax
import jax.numpy as jnp
from jax import lax
from jax.experimental import pallas as pl
from jax.experimental.pallas import tpu as pltpu
from jax.experimental.pallas import tpu_sc as plsc

VOCAB = 1_000_000
D = 16
EPS = 1e-5

B = 16384
H = 200

NC = 2
NS = 16
NW = NC * NS

BPW = B // NW
NBT = BPW // 128
HT = H // 8


_TC_BLOCK_W = 65536
VP = 1_048_576
VPX = VP // 128


_TC_BLOCK_K = _TC_BLOCK_W // 128


def _norm_body(t_ref, o_ref):
    tt = t_ref[...]
    m = jnp.mean(tt, axis=0, keepdims=True)
    d = tt - m
    v = jnp.mean(d * d, axis=0, keepdims=True)
    r = d * lax.rsqrt(v + EPS)
    o_ref[...] = r.reshape(2, 8, _TC_BLOCK_K, 128).swapaxes(1, 2)


def _normalize_table(table):
    grid = (VP + _TC_BLOCK_W - 1) // _TC_BLOCK_W
    return pl.pallas_call(
        _norm_body,
        grid=(grid,),
        in_specs=[pl.BlockSpec((D, _TC_BLOCK_W), lambda i: (0, i))],
        out_specs=pl.BlockSpec((2, _TC_BLOCK_K, 8, 128),
                               lambda i: (0, i, 0, 0)),
        out_shape=jax.ShapeDtypeStruct((2, VPX, 8, 128), jnp.float32),
    )(table.T)


VPT = VP // NW
TCH = 1024
NCH = VPT // TCH


def _make_prep():
    mesh = plsc.VectorSubcoreMesh(core_axis_name="c", subcore_axis_name="s",
                                  num_cores=NC, num_subcores=NS)

    @functools.partial(
        pl.kernel,
        out_type=jax.ShapeDtypeStruct((VP, D), jnp.float32),
        mesh=mesh,
        scratch_types=[
            pltpu.VMEM((2, 8, 8, 128), jnp.float32),
            pltpu.VMEM((2, 8, 8, 128), jnp.float32),
            pltpu.VMEM((TCH, 17), jnp.float32),
            pltpu.VMEM((TCH, 17), jnp.float32),
            pltpu.SemaphoreType.DMA,
            pltpu.SemaphoreType.DMA,
            pltpu.SemaphoreType.DMA,
            pltpu.SemaphoreType.DMA,
        ],
        compiler_params=pltpu.CompilerParams(use_tc_tiling_on_sc=False,
                                             needs_layout_passes=False),
    )
    def prep_k(ntp_hbm, out_hbm, in0, in1, ob0, ob1, si0, si1, so0, so1):
        wid = lax.axis_index("s") * NC + lax.axis_index("c")
        v0 = wid * VPT
        vt0 = v0 // 128
        in_b = (in0, in1)
        ob_b = (ob0, ob1)
        si_b = (si0, si1)
        so_b = (so0, so1)
        iota = lax.iota(jnp.int32, 16)
        col_d = [jnp.full((16,), 0, jnp.int32) + d for d in range(D)]

        def load(k, b):
            pltpu.async_copy(ntp_hbm.at[:, pl.ds(vt0 + k * 8, 8)],
                             in_b[b], si_b[b])

        def wait_load(b):
            pltpu.make_async_copy(ntp_hbm.at[:, pl.ds(0, 8)], in_b[b],
                                  si_b[b]).wait()

        def store(k, b):
            pltpu.async_copy(ob_b[b].at[:, pl.ds(0, D)],
                             out_hbm.at[pl.ds(v0 + k * TCH, TCH)], so_b[b])

        def wait_store(b):
            pltpu.make_async_copy(ob_b[b].at[:, pl.ds(0, D)],
                                  out_hbm.at[pl.ds(0, TCH)], so_b[b]).wait()

        def transpose_chunk(b):
            inb = in_b[b]
            ob = ob_b[b]

            def gbody(m, carry):
                vtl = lax.div(m, 8)
                g = lax.rem(m, 8)
                row_v = iota + m * 16
                for dt in range(2):
                    for ds in range(8):
                        v = inb[dt, vtl, ds, pl.ds(g * 16, 16)]
                        plsc.store_scatter(ob, [row_v, col_d[dt * 8 + ds]], v)
                return carry

            lax.fori_loop(0, TCH // 16, gbody, 0)

        load(0, 0)

        def body(kp, carry):
            for u in range(2):
                k = 2 * kp + u
                b = u
                wait_load(b)

                @pl.when(k < NCH - 1)
                def _():
                    load(k + 1, 1 - b)

                @pl.when(k >= 2)
                def _():
                    wait_store(b)

                transpose_chunk(b)
                store(k, b)
            return carry

        lax.fori_loop(0, NCH // 2, body, 0)
        wait_store(0)
        wait_store(1)

    return prep_k


_N_STEPS = H


def _make_gather():
    mesh = plsc.VectorSubcoreMesh(core_axis_name="c", subcore_axis_name="s",
                                  num_cores=NC, num_subcores=NS)

    @functools.partial(
        pl.kernel,
        out_type=jax.ShapeDtypeStruct((B * H * D // 128, 128), jnp.float32),
        mesh=mesh,
        scratch_types=[
            pltpu.VMEM((NBT * 8, 128), jnp.int32),
            pltpu.VMEM((NBT * 8, 128), jnp.int32),
            pltpu.SemaphoreType.DMA,
            pltpu.SemaphoreType.DMA,
            pltpu.VMEM((BPW, D), jnp.float32),
            pltpu.VMEM((BPW, D), jnp.float32),
            pltpu.VMEM((2, 40, 129), jnp.float32),
            pltpu.VMEM((2, 40, 129), jnp.float32),
            pltpu.SemaphoreType.DMA,
            pltpu.SemaphoreType.DMA,
            pltpu.SemaphoreType.DMA,
            pltpu.SemaphoreType.DMA,
        ],
        compiler_params=pltpu.CompilerParams(use_tc_tiling_on_sc=False,
                                             needs_layout_passes=False),
    )
    def gather_k(tab_hbm, q_hbm, p_hbm, idx0, idx1, qi0, qi1, rows0, rows1,
                 st0, st1, sg0, sg1, ss0, ss1):
        wid = lax.axis_index("s") * NC + lax.axis_index("c")
        idx_b = (idx0, idx1)
        qi_b = (qi0, qi1)
        rows_b = (rows0, rows1)
        stage_b = (st0, st1)
        sg_b = (sg0, sg1)
        ss_b = (ss0, ss1)

        iota = lax.iota(jnp.int32, 16)
        dt_vec = lax.shift_right_logical(iota, 3)
        ds_vec = lax.bitwise_and(iota, jnp.int32(7))

        def load_idx_block(ht, ib):
            pltpu.async_copy(q_hbm.at[pl.ds(ht * 1024 + wid * (NBT * 8),
                                            NBT * 8)], idx_b[ib], qi_b[ib])

        def wait_idx_block(ib):
            pltpu.make_async_copy(q_hbm.at[pl.ds(0, NBT * 8)], idx_b[ib],
                                  qi_b[ib]).wait()

        def fire(s, a, ib):
            hs1 = lax.rem(s, 8)
            idx_v = idx_b[ib]
            for c in range(NBT):
                pltpu.async_copy(tab_hbm.at[idx_v.at[c * 8 + hs1]],
                                 rows_b[a].at[pl.ds(c * 128, 128)], sg_b[a])

        def wait_gather(a):
            for c in range(NBT):
                pltpu.make_async_copy(tab_hbm.at[idx_b[0].at[0]],
                                      rows_b[a].at[pl.ds(c * 128, 128)],
                                      sg_b[a]).wait()

        def wait_store(a):
            for dt in range(2):
                pltpu.make_async_copy(
                    stage_b[a].at[dt, pl.ds(0, NBT * 8), pl.ds(0, 128)],
                    p_hbm.at[pl.ds(0, NBT * 8)],
                    ss_b[a]).wait()

        def transpose(a):
            rows_v = rows_b[a]
            stage = stage_b[a]
            for btl in range(NBT):
                row32 = ds_vec + btl * 8

                def tbody(g, carry):
                    base = btl * 128 + g * 8
                    for u in range(8):
                        r = base + u
                        v = rows_v[r, :]
                        bl_vec = jnp.full((16,), 0, jnp.int32) + (r - btl * 128)
                        plsc.store_scatter(stage, [dt_vec, row32, bl_vec], v)
                    return carry

                lax.fori_loop(0, 16, tbody, 0)

        def store(s, a):
            for dt in range(2):
                pltpu.async_copy(
                    stage_b[a].at[dt, pl.ds(0, NBT * 8), pl.ds(0, 128)],
                    p_hbm.at[pl.ds(s * 2048 + dt * 1024 + wid * (NBT * 8),
                                   NBT * 8)],
                    ss_b[a])

        def fire_sel(s, a, cur):
            @pl.when(cur == 0)
            def _():
                fire(s, a, 0)

            @pl.when(cur == 1)
            def _():
                fire(s, a, 1)

        load_idx_block(0, 0)
        wait_idx_block(0)
        fire(0, 0, 0)
        load_idx_block(1, 1)

        def body(k, carry):
            for u in range(2):
                s = 2 * k + u
                a = u
                hs = lax.rem(s, 8)
                ht = lax.div(s, 8)
                cur = lax.rem(ht, 2)

                @pl.when(jnp.logical_and(s < _N_STEPS - 1, hs != 7))
                def _():
                    fire_sel(s + 1, 1 - a, cur)

                wait_gather(a)

                @pl.when(jnp.logical_and(s < _N_STEPS - 1, hs == 7))
                def _():
                    @pl.when(cur == 0)
                    def _():
                        wait_idx_block(1)
                        fire(s + 1, 1 - a, 1)

                    @pl.when(cur == 1)
                    def _():
                        wait_idx_block(0)
                        fire(s + 1, 1 - a, 0)

                    @pl.when(ht + 2 < HT)
                    def _():
                        @pl.when(cur == 0)
                        def _():
                            load_idx_block(ht + 2, 0)

                        @pl.when(cur == 1)
                        def _():
                            load_idx_block(ht + 2, 1)

                @pl.when(s >= 2)
                def _():
                    wait_store(a)

                transpose(a)
                store(s, a)
            return carry

        lax.fori_loop(0, _N_STEPS // 2, body, 0)
        wait_store(0)
        wait_store(1)

    return gather_k


def kernel(x, table):
    ntab = _make_prep()(_normalize_table(table))
    q = (x.T.reshape(HT, 8, 128, 128).transpose(0, 2, 1, 3)
         .reshape(HT * 1024, 128))
    p = _make_gather()(ntab, q)
    out = (p.reshape(H, 2, 128, 8, 128).transpose(2, 4, 0, 1, 3)
           .reshape(B, H, D))
    return out
